# Initial kernel scaffold; baseline (speedup 1.0000x reference)
import jax
import jax.numpy as jnp
from jax.experimental import pallas as pl


def kernel(day, table):
    raise NotImplementedError("write your pallas kernel here")


# speedup vs baseline: 2.7862x; 2.7862x over previous
import functools

import jax
import jax.numpy as jnp
from jax import lax
from jax.experimental import pallas as pl
from jax.experimental.pallas import tpu as pltpu
from jax.experimental.pallas import tpu_sc as plsc

EMBED = 64
B_TOTAL = 16384 * 200
NUM_WORKERS = 32
PER_WORKER = B_TOTAL // NUM_WORKERS
GRP = 128
GPS = 8
STEP = GRP * GPS
STEPS = PER_WORKER // STEP


def _embed_kernel(table_hbm, idx_hbm, out_hbm, idx_v, rows_v, sem):
    wid = lax.axis_index("s") * 2 + lax.axis_index("c")
    row_base = wid * PER_WORKER
    grp_base = row_base // GRP

    def step(i, carry):
        g0 = pl.multiple_of(grp_base + i * GPS, 8)
        pltpu.sync_copy(idx_hbm.at[pl.ds(g0, GPS)], idx_v)
        copies = [
            pltpu.async_copy(
                table_hbm.at[idx_v.at[g]],
                rows_v.at[pl.ds(g * GRP, GRP)],
                sem,
            )
            for g in range(GPS)
        ]
        for cp in copies:
            cp.wait()
        pltpu.sync_copy(rows_v, out_hbm.at[pl.ds(row_base + i * STEP, STEP)])
        return carry

    lax.fori_loop(0, STEPS, step, 0)


@jax.jit
def kernel(day, table):
    idx2d = day.reshape(B_TOTAL // GRP, GRP).astype(jnp.int32)
    mesh = plsc.VectorSubcoreMesh(core_axis_name="c", subcore_axis_name="s")
    out = pl.kernel(
        _embed_kernel,
        mesh=mesh,
        compiler_params=pltpu.CompilerParams(use_tc_tiling_on_sc=False),
        out_type=jax.ShapeDtypeStruct((B_TOTAL, EMBED), jnp.float32),
        scratch_types=[
            pltpu.VMEM((GPS, GRP), jnp.int32),
            pltpu.VMEM((STEP, EMBED), jnp.float32),
            pltpu.SemaphoreType.DMA,
        ],
    )(table, idx2d)
    return out.reshape(day.shape[0], day.shape[1], EMBED)


# speedup vs baseline: 5.5940x; 2.0078x over previous
import jax
import jax.numpy as jnp
from jax import lax
from jax.experimental import pallas as pl
from jax.experimental.pallas import tpu as pltpu
from jax.experimental.pallas import tpu_sc as plsc

EMBED = 64
NUM_ROWS = 77
B_TOTAL = 16384 * 200
NUM_WORKERS = 32
PER_WORKER = B_TOTAL // NUM_WORKERS
GRP = 128
GPS = 4
STEP = GRP * GPS
NBUF = 2
STEPS = PER_WORKER // STEP


def _embed_kernel(table_hbm, idx_hbm, out_hbm, tab_sh, idx_v, rows_v,
                  gsem, osem0, osem1):
    cid = lax.axis_index("c")
    sid = lax.axis_index("s")
    wid = sid * 2 + cid
    row_base = wid * PER_WORKER
    grp_base = row_base // GRP
    osems = [osem0, osem1]

    @pl.when(sid == 0)
    def _stage():
        pltpu.sync_copy(table_hbm, tab_sh)

    plsc.subcore_barrier()

    @pl.loop(0, STEPS, step=NBUF)
    def _outer(i0):
        for b in range(NBUF):
            i = i0 + b

            @pl.when(i0 >= NBUF)
            def _reclaim():
                pltpu.make_async_copy(
                    rows_v.at[b], out_hbm.at[pl.ds(0, STEP)], osems[b]
                ).wait()

            g0 = pl.multiple_of(grp_base + i * GPS, GPS)
            pltpu.sync_copy(idx_hbm.at[pl.ds(g0, GPS)], idx_v.at[b])
            gathers = [
                pltpu.async_copy(
                    tab_sh.at[idx_v.at[b].at[g]],
                    rows_v.at[b].at[pl.ds(g * GRP, GRP)],
                    gsem,
                )
                for g in range(GPS)
            ]
            for cp in gathers:
                cp.wait()
            out_off = pl.multiple_of(row_base + i * STEP, STEP)
            pltpu.async_copy(rows_v.at[b], out_hbm.at[pl.ds(out_off, STEP)],
                             osems[b])

    for b in range(NBUF):
        pltpu.make_async_copy(
            rows_v.at[b], out_hbm.at[pl.ds(0, STEP)], osems[b]
        ).wait()


@jax.jit
def kernel(day, table):
    idx2d = day.reshape(B_TOTAL // GRP, GRP).astype(jnp.int32)
    mesh = plsc.VectorSubcoreMesh(core_axis_name="c", subcore_axis_name="s")
    out = pl.kernel(
        _embed_kernel,
        mesh=mesh,
        compiler_params=pltpu.CompilerParams(use_tc_tiling_on_sc=False),
        out_type=jax.ShapeDtypeStruct((B_TOTAL, EMBED), jnp.float32),
        scratch_types=[
            pltpu.VMEM_SHARED((NUM_ROWS, EMBED), jnp.float32),
            pltpu.VMEM((NBUF, GPS, GRP), jnp.int32),
            pltpu.VMEM((NBUF, STEP, EMBED), jnp.float32),
            pltpu.SemaphoreType.DMA,
            pltpu.SemaphoreType.DMA,
            pltpu.SemaphoreType.DMA,
        ],
    )(table, idx2d)
    return out.reshape(day.shape[0], day.shape[1], EMBED)


# speedup vs baseline: 5.8067x; 1.0380x over previous
import jax
import jax.numpy as jnp
from jax import lax
from jax.experimental import pallas as pl
from jax.experimental.pallas import tpu as pltpu
from jax.experimental.pallas import tpu_sc as plsc

EMBED = 64
NUM_ROWS = 77
B_TOTAL = 16384 * 200
NUM_WORKERS = 32
PER_WORKER = B_TOTAL // NUM_WORKERS
STEP = 512
NBUF = 2
STEPS = PER_WORKER // STEP


def _embed_kernel(table_hbm, idx_hbm, out_hbm, tab_sh, idx_v, rows_v,
                  gsem, osem0, osem1, isem0, isem1):
    cid = lax.axis_index("c")
    sid = lax.axis_index("s")
    wid = sid * 2 + cid
    row_base = wid * PER_WORKER
    osems = [osem0, osem1]
    isems = [isem0, isem1]

    def idx_slice(i):
        return idx_hbm.at[pl.ds(pl.multiple_of(row_base + i * STEP, STEP), STEP)]

    def out_slice(i):
        return out_hbm.at[pl.ds(pl.multiple_of(row_base + i * STEP, STEP), STEP)]

    @pl.when(sid == 0)
    def _stage():
        pltpu.sync_copy(table_hbm, tab_sh)

    plsc.subcore_barrier()

    pltpu.async_copy(idx_slice(0), idx_v.at[0], isems[0])

    @pl.loop(0, STEPS, step=NBUF)
    def _outer(i0):
        for b in range(NBUF):
            i = i0 + b
            nb = (b + 1) % NBUF

            pltpu.make_async_copy(idx_slice(0), idx_v.at[b], isems[b]).wait()

            @pl.when(i + 1 < STEPS)
            def _prefetch():
                pltpu.async_copy(idx_slice(i + 1), idx_v.at[nb], isems[nb])

            @pl.when(i0 >= NBUF)
            def _reclaim():
                pltpu.make_async_copy(
                    rows_v.at[b], out_slice(0), osems[b]
                ).wait()

            pltpu.async_copy(
                tab_sh.at[idx_v.at[b]], rows_v.at[b], gsem
            ).wait()

            pltpu.async_copy(rows_v.at[b], out_slice(i), osems[b])

    for b in range(NBUF):
        pltpu.make_async_copy(rows_v.at[b], out_slice(0), osems[b]).wait()


@jax.jit
def kernel(day, table):
    idx1d = day.reshape(B_TOTAL).astype(jnp.int32)
    mesh = plsc.VectorSubcoreMesh(core_axis_name="c", subcore_axis_name="s")
    out = pl.kernel(
        _embed_kernel,
        mesh=mesh,
        compiler_params=pltpu.CompilerParams(use_tc_tiling_on_sc=False),
        out_type=jax.ShapeDtypeStruct((B_TOTAL, EMBED), jnp.float32),
        scratch_types=[
            pltpu.VMEM_SHARED((NUM_ROWS, EMBED), jnp.float32),
            pltpu.VMEM((NBUF, STEP), jnp.int32),
            pltpu.VMEM((NBUF, STEP, EMBED), jnp.float32),
            pltpu.SemaphoreType.DMA,
            pltpu.SemaphoreType.DMA,
            pltpu.SemaphoreType.DMA,
            pltpu.SemaphoreType.DMA,
            pltpu.SemaphoreType.DMA,
        ],
    )(table, idx1d)
    return out.reshape(day.shape[0], day.shape[1], EMBED)
